# baseline (speedup 1.0000x reference)
import functools

import jax
import jax.numpy as jnp
from jax import lax
from jax.experimental import pallas as pl
from jax.experimental.pallas import tpu as pltpu
from jax.experimental.pallas import tpu_sc as plsc

B, C, H, W = 32, 512, 32, 32
S = 512
NC, NS, L = 2, 16, 16
NW = NC * NS
P = 128
NG = P // L
CK = 64
NCH = C // CK
TASKS = B * 8
TPW = TASKS // NW
STEPS = (256, 128, 64, 32, 16, 8, 4, 2)


def _body(in_hbm, rv_hbm, out_hbm, cs_ref, in0, in1, r0, r1, o0, o1,
          tot_ref, sem_in, sem_rv, sem_out):
    wid = lax.axis_index("s") * NC + lax.axis_index("c")
    lane = lax.broadcasted_iota(jnp.int32, (L,), 0)
    ins = (in0, in1)
    rs = (r0, r1)
    os_ = (o0, o1)

    def task(t, carry):
        b = t // 8
        j = t % 8

        for h in range(2):
            pltpu.async_copy(rv_hbm.at[b, pl.ds(h * CK, CK), j, :],
                             rs[h], sem_rv)

        def cs_pair(i, accs):
            for h in range(2):
                cc = i * 2 + h
                buf = ins[h]
                pltpu.make_async_copy(
                    in_hbm.at[b, pl.ds(cc * CK, CK), j, :], buf,
                    sem_in).wait()

                def csum(c_loc, accs, _buf=buf, _cc=cc):
                    base = (_cc * CK + c_loc) * P
                    new = []
                    for g in range(NG):
                        a = accs[g] + _buf[c_loc, pl.ds(g * L, L)]
                        cs_ref[pl.ds(base + g * L, L)] = a
                        new.append(a)
                    return tuple(new)

                accs = lax.fori_loop(0, CK, csum, accs)

                @pl.when(cc + 2 < NCH)
                def _(_buf=buf, _cc=cc):
                    pltpu.async_copy(
                        in_hbm.at[b, pl.ds((_cc + 2) * CK, CK), j, :],
                        _buf, sem_in)

            return accs

        zero = (jnp.zeros((L,), jnp.float32),) * NG
        totals = lax.fori_loop(0, NCH // 2, cs_pair, zero)
        for g in range(NG):
            tot_ref[pl.ds(g * L, L)] = totals[g]

        @pl.when(t + 1 < (wid + 1) * TPW)
        def _():
            nb = (t + 1) // 8
            nj = (t + 1) % 8
            for h in range(2):
                pltpu.async_copy(in_hbm.at[nb, pl.ds(h * CK, CK), nj, :],
                                 ins[h], sem_in)

        c255s = tuple(cs_ref[pl.ds(255 * P + g * L, L)] for g in range(NG))
        c127s = tuple(cs_ref[pl.ds(127 * P + g * L, L)] for g in range(NG))
        c383s = tuple(cs_ref[pl.ds(383 * P + g * L, L)] for g in range(NG))

        def q_pair(i, carry):
            for h in range(2):
                sc_ = i * 2 + h
                rb = rs[h]
                ob = os_[h]
                pltpu.make_async_copy(
                    rv_hbm.at[b, pl.ds(sc_ * CK, CK), j, :], rb,
                    sem_rv).wait()

                @pl.when(sc_ >= 2)
                def _(_ob=ob, _sc=sc_):
                    pltpu.make_async_copy(
                        _ob, out_hbm.at[b, pl.ds((_sc - 2) * CK, CK), j, :],
                        sem_out).wait()

                @plsc.parallel_loop(0, CK, step=1)
                def query(s_loc, _rb=rb, _ob=ob):
                    for g in range(NG):
                        t_val = _rb[s_loc, pl.ds(g * L, L)] * tot_ref[
                            pl.ds(g * L, L)]
                        d1 = c255s[g] < t_val
                        q = lane + (g * L + 255 * P) + jnp.where(
                            d1, 128 * P, -(128 * P))
                        v2 = jnp.where(d1, c383s[g], c127s[g])
                        q = q + jnp.where(v2 < t_val, 64 * P, -(64 * P))
                        for k in STEPS[2:]:
                            gv = plsc.load_gather(cs_ref, [q])
                            q = q + jnp.where(gv < t_val, (k // 2) * P,
                                              -(k // 2) * P)
                        gv = plsc.load_gather(cs_ref, [q])
                        pos = lax.shift_right_logical(q, 7) + jnp.where(
                            gv < t_val, 1, 0)
                        _ob[s_loc, pl.ds(g * L, L)] = pos

                pltpu.async_copy(
                    ob, out_hbm.at[b, pl.ds(sc_ * CK, CK), j, :], sem_out)

                @pl.when(sc_ + 2 < NCH)
                def _(_rb=rb, _sc=sc_):
                    pltpu.async_copy(
                        rv_hbm.at[b, pl.ds((_sc + 2) * CK, CK), j, :],
                        _rb, sem_rv)

            return 0

        lax.fori_loop(0, NCH // 2, q_pair, 0)

        for sc_ in (NCH - 2, NCH - 1):
            pltpu.make_async_copy(
                os_[sc_ & 1], out_hbm.at[b, pl.ds(sc_ * CK, CK), j, :],
                sem_out).wait()
        return 0

    t0 = wid * TPW
    b0 = t0 // 8
    j0 = t0 % 8
    for h in range(2):
        pltpu.async_copy(in_hbm.at[b0, pl.ds(h * CK, CK), j0, :],
                         ins[h], sem_in)
    lax.fori_loop(t0, (wid + 1) * TPW, task, 0)


@jax.jit
def kernel(input, random_values):
    mesh = plsc.VectorSubcoreMesh(core_axis_name="c", subcore_axis_name="s")
    x = input.reshape(B, C, 8, P)
    rv = random_values.reshape(B, S, 8, P)
    spikes = pl.kernel(
        _body,
        out_type=jax.ShapeDtypeStruct((B, S, 8, P), jnp.int32),
        mesh=mesh,
        compiler_params=pltpu.CompilerParams(
            needs_layout_passes=False, use_tc_tiling_on_sc=True
        ),
        scratch_types=[
            pltpu.VMEM((C * P,), jnp.float32),
            pltpu.VMEM((CK, P), jnp.float32),
            pltpu.VMEM((CK, P), jnp.float32),
            pltpu.VMEM((CK, P), jnp.float32),
            pltpu.VMEM((CK, P), jnp.float32),
            pltpu.VMEM((CK, P), jnp.int32),
            pltpu.VMEM((CK, P), jnp.int32),
            pltpu.VMEM((P,), jnp.float32),
            pltpu.SemaphoreType.DMA,
            pltpu.SemaphoreType.DMA,
            pltpu.SemaphoreType.DMA,
        ],
    )(x, rv)
    return spikes.reshape(B, S, H, W).astype(jnp.int64)


# speedup vs baseline: 1.0687x; 1.0687x over previous
import functools

import jax
import jax.numpy as jnp
from jax import lax
from jax.experimental import pallas as pl
from jax.experimental.pallas import tpu as pltpu
from jax.experimental.pallas import tpu_sc as plsc

B, C, H, W = 32, 512, 32, 32
S = 512
NC, NS, L = 2, 16, 16
NW = NC * NS
P = 128
NG = P // L
CK = 64
NCH = C // CK
TASKS = B * 8
TPW = TASKS // NW
STEPS = (256, 128, 64, 32, 16, 8, 4, 2)


def _body(in_hbm, rv_hbm, out_hbm, cs_ref, in0, in1, r0, r1, o0, o1,
          sem_in, sem_rv, sem_out):
    wid = lax.axis_index("s") * NC + lax.axis_index("c")
    lane = lax.broadcasted_iota(jnp.int32, (L,), 0)
    ins = (in0, in1)
    rs = (r0, r1)
    os_ = (o0, o1)

    def task(t, carry):
        b = t // 8
        j = t % 8

        for h in range(2):
            pltpu.async_copy(rv_hbm.at[b, pl.ds(h * CK, CK), j, :],
                             rs[h], sem_rv)

        def cs_pair(i, accs):
            for h in range(2):
                cc = i * 2 + h
                buf = ins[h]
                pltpu.make_async_copy(
                    in_hbm.at[b, pl.ds(cc * CK, CK), j, :], buf,
                    sem_in).wait()

                def csum(c_loc, accs, _buf=buf, _cc=cc):
                    base = (_cc * CK + c_loc) * P
                    new = []
                    for g in range(NG):
                        a = accs[g] + _buf[c_loc, pl.ds(g * L, L)]
                        cs_ref[pl.ds(base + g * L, L)] = a
                        new.append(a)
                    return tuple(new)

                accs = lax.fori_loop(0, CK, csum, accs)

                @pl.when(cc + 2 < NCH)
                def _(_buf=buf, _cc=cc):
                    pltpu.async_copy(
                        in_hbm.at[b, pl.ds((_cc + 2) * CK, CK), j, :],
                        _buf, sem_in)

            return accs

        zero = (jnp.zeros((L,), jnp.float32),) * NG
        totals = lax.fori_loop(0, NCH // 2, cs_pair, zero)

        @pl.when(t + 1 < (wid + 1) * TPW)
        def _():
            nb = (t + 1) // 8
            nj = (t + 1) % 8
            for h in range(2):
                pltpu.async_copy(in_hbm.at[nb, pl.ds(h * CK, CK), nj, :],
                                 ins[h], sem_in)

        c255s = tuple(cs_ref[pl.ds(255 * P + g * L, L)] for g in range(NG))
        c127s = tuple(cs_ref[pl.ds(127 * P + g * L, L)] for g in range(NG))
        c383s = tuple(cs_ref[pl.ds(383 * P + g * L, L)] for g in range(NG))

        def q_pair(i, carry):
            for h in range(2):
                sc_ = i * 2 + h
                rb = rs[h]
                ob = os_[h]
                pltpu.make_async_copy(
                    rv_hbm.at[b, pl.ds(sc_ * CK, CK), j, :], rb,
                    sem_rv).wait()

                @pl.when(sc_ >= 2)
                def _(_ob=ob, _sc=sc_):
                    pltpu.make_async_copy(
                        _ob, out_hbm.at[b, pl.ds((_sc - 2) * CK, CK), j, :],
                        sem_out).wait()

                @plsc.parallel_loop(0, CK, step=1)
                def scale(s_loc, _rb=rb):
                    for g in range(NG):
                        sl = pl.ds(g * L, L)
                        _rb[s_loc, sl] = _rb[s_loc, sl] * totals[g]

                @plsc.parallel_loop(0, CK, step=1)
                def query(s_loc, _rb=rb, _ob=ob):
                    for g in range(NG):
                        t_val = _rb[s_loc, pl.ds(g * L, L)]
                        d1 = c255s[g] < t_val
                        q = lane + (g * L + 255 * P) + jnp.where(
                            d1, 128 * P, -(128 * P))
                        v2 = jnp.where(d1, c383s[g], c127s[g])
                        q = q + jnp.where(v2 < t_val, 64 * P, -(64 * P))
                        for k in STEPS[2:]:
                            gv = plsc.load_gather(cs_ref, [q])
                            q = q + jnp.where(gv < t_val, (k // 2) * P,
                                              -(k // 2) * P)
                        gv = plsc.load_gather(cs_ref, [q])
                        pos = lax.shift_right_logical(q, 7) + jnp.where(
                            gv < t_val, 1, 0)
                        _ob[s_loc, pl.ds(g * L, L)] = pos

                pltpu.async_copy(
                    ob, out_hbm.at[b, pl.ds(sc_ * CK, CK), j, :], sem_out)

                @pl.when(sc_ + 2 < NCH)
                def _(_rb=rb, _sc=sc_):
                    pltpu.async_copy(
                        rv_hbm.at[b, pl.ds((_sc + 2) * CK, CK), j, :],
                        _rb, sem_rv)

            return 0

        lax.fori_loop(0, NCH // 2, q_pair, 0)

        for sc_ in (NCH - 2, NCH - 1):
            pltpu.make_async_copy(
                os_[sc_ & 1], out_hbm.at[b, pl.ds(sc_ * CK, CK), j, :],
                sem_out).wait()
        return 0

    t0 = wid * TPW
    b0 = t0 // 8
    j0 = t0 % 8
    for h in range(2):
        pltpu.async_copy(in_hbm.at[b0, pl.ds(h * CK, CK), j0, :],
                         ins[h], sem_in)
    lax.fori_loop(t0, (wid + 1) * TPW, task, 0)


@jax.jit
def kernel(input, random_values):
    mesh = plsc.VectorSubcoreMesh(core_axis_name="c", subcore_axis_name="s")
    x = input.reshape(B, C, 8, P)
    rv = random_values.reshape(B, S, 8, P)
    spikes = pl.kernel(
        _body,
        out_type=jax.ShapeDtypeStruct((B, S, 8, P), jnp.int32),
        mesh=mesh,
        compiler_params=pltpu.CompilerParams(
            needs_layout_passes=False, use_tc_tiling_on_sc=True
        ),
        scratch_types=[
            pltpu.VMEM((C * P,), jnp.float32),
            pltpu.VMEM((CK, P), jnp.float32),
            pltpu.VMEM((CK, P), jnp.float32),
            pltpu.VMEM((CK, P), jnp.float32),
            pltpu.VMEM((CK, P), jnp.float32),
            pltpu.VMEM((CK, P), jnp.int32),
            pltpu.VMEM((CK, P), jnp.int32),
            pltpu.SemaphoreType.DMA,
            pltpu.SemaphoreType.DMA,
            pltpu.SemaphoreType.DMA,
        ],
    )(x, rv)
    return spikes.reshape(B, S, H, W).astype(jnp.int64)
